# Initial kernel scaffold; baseline (speedup 1.0000x reference)
import jax
import jax.numpy as jnp
from jax.experimental import pallas as pl


def kernel(preds, gt_masks):
    raise NotImplementedError("write your pallas kernel here")


# speedup vs baseline: 20.5642x; 20.5642x over previous
import jax
import jax.numpy as jnp
from jax.experimental import pallas as pl
from jax.experimental.pallas import tpu as pltpu

_N = 2097152
_K = 209715
_ROWS = 2048
_COLS = 1024
_CHUNKS = 8
_CROWS = _ROWS // _CHUNKS
_HI_BITS = 0x42C80000
_SEARCH_ITERS = 31


def _body(p_ref, t_ref, out_ref, bits_ref, acc_ref):
    i = pl.program_id(0)

    @pl.when(i == 0)
    def _init():
        acc_ref[0] = 0.0
        acc_ref[1] = 0.0
        acc_ref[2] = 0.0

    p = p_ref[...]
    t = t_ref[...]
    acc_ref[0] += jnp.sum(p)
    acc_ref[1] += jnp.sum(t)
    acc_ref[2] += jnp.sum(p * t)

    log_p = jnp.maximum(jnp.log(p), -100.0)
    log_1mp = jnp.maximum(jnp.log1p(-p), -100.0)
    bce = jnp.maximum(-(t * log_p + (1.0 - t) * log_1mp), 0.0)
    bits_ref[pl.ds(i * _CROWS, _CROWS), :] = pltpu.bitcast(bce, jnp.int32)

    @pl.when(i == _CHUNKS - 1)
    def _select():
        def step(_, carry):
            lo, hi = carry
            mid = lo + (hi - lo + 1) // 2
            c = jnp.sum((bits_ref[...] >= mid).astype(jnp.int32))
            big = c >= _K
            lo = jnp.where(big, mid, lo)
            hi = jnp.where(big, hi, mid - 1)
            return lo, hi

        lo, _ = jax.lax.fori_loop(
            0, _SEARCH_ITERS, step, (jnp.int32(0), jnp.int32(_HI_BITS))
        )

        bits = bits_ref[...]
        b = pltpu.bitcast(bits, jnp.float32)
        vk = jnp.max(jnp.where(bits == lo, b, 0.0))
        gt = bits > lo
        c_gt = jnp.sum(gt.astype(jnp.int32))
        s_gt = jnp.sum(jnp.where(gt, b, 0.0))
        topk_mean = (s_gt + (_K - c_gt).astype(jnp.float32) * vk) / _K
        dice = 1.0 - (2.0 * acc_ref[2] + 1.0) / (acc_ref[0] + acc_ref[1] + 1.0)
        out_ref[...] = (dice + topk_mean).reshape(1, 1)


def kernel(preds, gt_masks):
    p = preds.reshape(_ROWS, _COLS)
    t = gt_masks.reshape(_ROWS, _COLS)
    out = pl.pallas_call(
        _body,
        grid=(_CHUNKS,),
        in_specs=[
            pl.BlockSpec((_CROWS, _COLS), lambda i: (i, 0)),
            pl.BlockSpec((_CROWS, _COLS), lambda i: (i, 0)),
        ],
        out_specs=pl.BlockSpec((1, 1), lambda i: (0, 0)),
        out_shape=jax.ShapeDtypeStruct((1, 1), jnp.float32),
        scratch_shapes=[
            pltpu.VMEM((_ROWS, _COLS), jnp.int32),
            pltpu.SMEM((4,), jnp.float32),
        ],
    )(p, t)
    return out[0, 0]
